# baseline (speedup 1.0000x reference)
import functools

import jax
import jax.numpy as jnp
from jax import lax
from jax.experimental import pallas as pl
from jax.experimental.pallas import tpu as pltpu
from jax.experimental.pallas import tpu_sc as plsc

BATCH = 16384
DIM = 64
LANES = 16


def _emb_mul_body(table_hbm, lat_hbm, idxk_hbm, off_hbm, out_hbm,
                  off_sh, off_s, k_v, rows_v, lat_v, gsem, lsem, nc):
    wid = lax.axis_index("s") * nc + lax.axis_index("c")
    b_per_w = BATCH // (nc * 16)
    base = pl.multiple_of(wid * b_per_w, b_per_w)
    base2 = pl.multiple_of(wid * (b_per_w // 2), b_per_w // 2)
    rows2 = b_per_w // 2

    pltpu.sync_copy(off_hbm.at[pl.ds(base, b_per_w)], off_sh.at[wid])
    pltpu.sync_copy(off_sh.at[wid], off_s)
    pltpu.sync_copy(idxk_hbm.at[pl.ds(base, b_per_w)], k_v)

    latcp = pltpu.async_copy(lat_hbm.at[pl.ds(base2, rows2)], lat_v, lsem)
    pltpu.async_copy(table_hbm.at[k_v], rows_v, gsem).wait()
    latcp.wait()

    def mul(p, _):
        for u in range(2):
            i = p * 2 + u
            off = off_s[i]
            src = rows_v.at[i]
            for j in range(DIM // LANES):
                dsl = pl.ds(u * DIM + j * LANES, LANES)
                lat_v[p, dsl] = lat_v[p, dsl] * src[pl.ds(off + j * LANES,
                                                          LANES)]
        return 0

    lax.fori_loop(0, rows2, mul, 0)
    pltpu.sync_copy(lat_v, out_hbm.at[pl.ds(base2, rows2)])


def kernel(latent, label, emb_table):
    info = plsc.get_sparse_core_info()
    nc = info.num_cores
    b_per_w = BATCH // (nc * info.num_subcores)
    mesh = plsc.VectorSubcoreMesh(core_axis_name="c", subcore_axis_name="s")
    fn = pl.kernel(
        functools.partial(_emb_mul_body, nc=nc),
        mesh=mesh,
        out_type=jax.ShapeDtypeStruct((BATCH // 2, 2 * DIM), jnp.float32),
        scratch_types=[
            pltpu.VMEM_SHARED((32, b_per_w), jnp.int32),
            pltpu.SMEM((b_per_w,), jnp.int32),
            pltpu.VMEM((b_per_w,), jnp.int32),
            pltpu.VMEM((b_per_w, 2 * DIM), jnp.float32),
            pltpu.VMEM((b_per_w // 2, 2 * DIM), jnp.float32),
            pltpu.SemaphoreType.DMA,
            pltpu.SemaphoreType.DMA,
        ],
    )
    label = label.astype(jnp.int32)
    table2 = emb_table.reshape(emb_table.shape[0] // 2, 2 * DIM)
    lat2 = latent.reshape(BATCH // 2, 2 * DIM)
    idx_k = lax.shift_right_logical(label, 1)
    idx_off = (label & 1) * DIM
    out2 = fn(table2, lat2, idx_k, idx_off)
    return out2.reshape(BATCH, DIM)


# speedup vs baseline: 1.0348x; 1.0348x over previous
import functools

import jax
import jax.numpy as jnp
from jax import lax
from jax.experimental import pallas as pl
from jax.experimental.pallas import tpu as pltpu
from jax.experimental.pallas import tpu_sc as plsc

BATCH = 16384
DIM = 64
LANES = 16
RELAYOUT_ROWS = 5000


def _relayout_body(a_ref, b_ref, o_ref):
    o_ref[:, 0:DIM] = a_ref[...]
    o_ref[:, DIM:2 * DIM] = b_ref[...]


def _relayout_tc(table):
    n = table.shape[0]
    half = n // 2
    grid = half // RELAYOUT_ROWS
    return pl.pallas_call(
        _relayout_body,
        grid=(grid,),
        in_specs=[
            pl.BlockSpec((RELAYOUT_ROWS, DIM), lambda i: (i, 0)),
            pl.BlockSpec((RELAYOUT_ROWS, DIM),
                         lambda i: (i + half // RELAYOUT_ROWS, 0)),
        ],
        out_specs=pl.BlockSpec((RELAYOUT_ROWS, 2 * DIM), lambda i: (i, 0)),
        out_shape=jax.ShapeDtypeStruct((half, 2 * DIM), jnp.float32),
    )(table, table)


def _emb_mul_body(table_hbm, lat_hbm, idxk_hbm, off_hbm, out_hbm,
                  off_sh, off_s, k_v, rows_v, lat_v, gsem, lsem, nc):
    wid = lax.axis_index("s") * nc + lax.axis_index("c")
    b_per_w = BATCH // (nc * 16)
    base = pl.multiple_of(wid * b_per_w, b_per_w)

    pltpu.sync_copy(off_hbm.at[pl.ds(base, b_per_w)], off_sh.at[wid])
    pltpu.sync_copy(off_sh.at[wid], off_s)
    pltpu.sync_copy(idxk_hbm.at[pl.ds(base, b_per_w)], k_v)

    chunk = b_per_w // 2
    for r in range(2):
        cbase = pl.multiple_of(base + r * chunk, chunk)
        latcp = pltpu.async_copy(lat_hbm.at[pl.ds(cbase, chunk)], lat_v, lsem)
        pltpu.async_copy(table_hbm.at[k_v.at[pl.ds(r * chunk, chunk)]],
                         rows_v, gsem).wait()
        latcp.wait()

        def mul(i, _):
            off = off_s[r * chunk + i]
            src = rows_v.at[i]
            for j in range(DIM // LANES):
                dsl = pl.ds(j * LANES, LANES)
                lat_v[i, dsl] = lat_v[i, dsl] * src[pl.ds(off + j * LANES,
                                                          LANES)]
            return 0

        lax.fori_loop(0, chunk, mul, 0)
        pltpu.sync_copy(lat_v, out_hbm.at[pl.ds(cbase, chunk)])


def kernel(latent, label, emb_table):
    info = plsc.get_sparse_core_info()
    nc = info.num_cores
    b_per_w = BATCH // (nc * info.num_subcores)
    mesh = plsc.VectorSubcoreMesh(core_axis_name="c", subcore_axis_name="s")
    fn = pl.kernel(
        functools.partial(_emb_mul_body, nc=nc),
        mesh=mesh,
        out_type=jax.ShapeDtypeStruct((BATCH, DIM), jnp.float32),
        scratch_types=[
            pltpu.VMEM_SHARED((32, b_per_w), jnp.int32),
            pltpu.SMEM((b_per_w,), jnp.int32),
            pltpu.VMEM((b_per_w,), jnp.int32),
            pltpu.VMEM((b_per_w // 2, 2 * DIM), jnp.float32),
            pltpu.VMEM((b_per_w // 2, DIM), jnp.float32),
            pltpu.SemaphoreType.DMA,
            pltpu.SemaphoreType.DMA,
        ],
    )
    label = label.astype(jnp.int32)
    table2 = _relayout_tc(emb_table)
    half = emb_table.shape[0] // 2
    in_hi = (label >= half).astype(jnp.int32)
    idx_k = label - in_hi * half
    idx_off = in_hi * DIM
    return fn(table2, latent, idx_k, idx_off)
